# Initial kernel scaffold; baseline (speedup 1.0000x reference)
import jax
import jax.numpy as jnp
from jax.experimental import pallas as pl


def kernel(feat, edge, ep, fc_w, fc_b, proj_w, proj_b, gcn_w):
    raise NotImplementedError("write your pallas kernel here")


# speedup vs baseline: 1.0371x; 1.0371x over previous
import math

import jax
import jax.numpy as jnp
from jax.experimental import pallas as pl

N = 4096
DIM = 128
KSEL = 2048
ALPHA = 0.1
LAMDA = 0.5

_INTERPRET = False


def _select_kernel(s_ref, h_ref, newh_ref, mask_ref):
    s = s_ref[...]
    u = jax.lax.bitcast_convert_type(s, jnp.int32)

    def body_t(i, t):
        b = 30 - i
        cand = t | (jnp.int32(1) << b)
        cnt = jnp.sum((u >= cand).astype(jnp.int32))
        return jax.lax.select(cnt >= KSEL, cand, t)

    t = jax.lax.fori_loop(0, 31, body_t, jnp.int32(0))
    cnt_gt = jnp.sum((u > t).astype(jnp.int32))
    need = KSEL - cnt_gt
    eq = u == t
    gidx = jax.lax.broadcasted_iota(jnp.int32, s.shape, 0)

    def body_c(i, c):
        b = 12 - i
        cand = c | (jnp.int32(1) << b)
        f = jnp.sum((eq & (gidx < cand)).astype(jnp.int32))
        return jax.lax.select(f <= need, cand, c)

    c = jax.lax.fori_loop(0, 13, body_c, jnp.int32(0))
    m = (u > t) | (eq & (gidx < c))
    mf = m.astype(jnp.float32)
    mask_ref[...] = mf
    newh_ref[...] = h_ref[...] * (s * mf)


def _binmask_kernel(e_ref, mr_ref, mc_ref, ar_ref, ac_ref):
    nz = e_ref[...] != 0.0
    ar_ref[...] = jnp.where(nz, mr_ref[...], 0.0).astype(jnp.bfloat16)
    ac_ref[...] = jnp.where(nz, mc_ref[...], 0.0).astype(jnp.bfloat16)


def _sq_kernel(l_ref, r_ref, g_ref):
    acc = jax.lax.dot_general(
        l_ref[...], r_ref[...], (((1,), (0,)), ((), ())),
        preferred_element_type=jnp.float32)
    g_ref[...] = (acc != 0.0).astype(jnp.bfloat16)


def _gcn_kernel(theta, g_ref, hfull_ref, hrow_ref, h0row_ref, w_ref, out_ref):
    h = hfull_ref[...]
    ha = h.astype(jnp.bfloat16)
    hb = (h - ha.astype(jnp.float32)).astype(jnp.bfloat16)
    g = g_ref[...]
    dn = (((1,), (0,)), ((), ()))
    hi = (jax.lax.dot_general(g, ha, dn, preferred_element_type=jnp.float32)
          + jax.lax.dot_general(g, hb, dn, preferred_element_type=jnp.float32))
    w = w_ref[...]
    h0r = h0row_ref[...]
    hr = hrow_ref[...]
    sup = (jax.lax.dot_general(hi, w[:DIM], dn, preferred_element_type=jnp.float32)
           + jax.lax.dot_general(h0r, w[DIM:], dn, preferred_element_type=jnp.float32))
    r = (1.0 - ALPHA) * hi + ALPHA * h0r
    out = theta * sup + (1.0 - theta) * r + hr
    out_ref[...] = jnp.maximum(hr + out, 0.0)


def kernel(feat, edge, ep, fc_w, fc_b, proj_w, proj_b, gcn_w):
    f32 = jnp.float32
    bf16 = jnp.bfloat16

    h = jax.nn.relu(feat @ fc_w.T + fc_b)
    weights = (h @ proj_w.T + proj_b).squeeze()
    scores = jax.nn.sigmoid(weights).reshape(N, 1)

    new_h, mask = pl.pallas_call(
        _select_kernel,
        out_shape=(jax.ShapeDtypeStruct((N, DIM), f32),
                   jax.ShapeDtypeStruct((N, 1), f32)),
        interpret=_INTERPRET,
    )(scores, h)
    mask_row = mask.reshape(1, N)

    BM_A = 512
    ar, ac = pl.pallas_call(
        _binmask_kernel,
        grid=(N // BM_A,),
        in_specs=[
            pl.BlockSpec((BM_A, N), lambda i: (i, 0)),
            pl.BlockSpec((BM_A, 1), lambda i: (i, 0)),
            pl.BlockSpec((1, N), lambda i: (0, 0)),
        ],
        out_specs=(pl.BlockSpec((BM_A, N), lambda i: (i, 0)),
                   pl.BlockSpec((BM_A, N), lambda i: (i, 0))),
        out_shape=(jax.ShapeDtypeStruct((N, N), bf16),
                   jax.ShapeDtypeStruct((N, N), bf16)),
        interpret=_INTERPRET,
    )(edge, mask, mask_row)

    BM, BN = 1024, 1024
    g = pl.pallas_call(
        _sq_kernel,
        grid=(N // BM, N // BN),
        in_specs=[
            pl.BlockSpec((BM, N), lambda i, j: (i, 0)),
            pl.BlockSpec((N, BN), lambda i, j: (0, j)),
        ],
        out_specs=pl.BlockSpec((BM, BN), lambda i, j: (i, j)),
        out_shape=jax.ShapeDtypeStruct((N, N), bf16),
        interpret=_INTERPRET,
    )(ar, ac)

    BMG = 1024
    hcur = new_h
    for layer in range(1, 4):
        theta = math.log(LAMDA / layer + 1.0)
        hcur = pl.pallas_call(
            lambda g_ref, hf_ref, hr_ref, h0_ref, w_ref, o_ref, _t=theta: _gcn_kernel(
                _t, g_ref, hf_ref, hr_ref, h0_ref, w_ref, o_ref),
            grid=(N // BMG,),
            in_specs=[
                pl.BlockSpec((BMG, N), lambda i: (i, 0)),
                pl.BlockSpec((N, DIM), lambda i: (0, 0)),
                pl.BlockSpec((BMG, DIM), lambda i: (i, 0)),
                pl.BlockSpec((BMG, DIM), lambda i: (i, 0)),
                pl.BlockSpec((2 * DIM, DIM), lambda i: (0, 0)),
            ],
            out_specs=pl.BlockSpec((BMG, DIM), lambda i: (i, 0)),
            out_shape=jax.ShapeDtypeStruct((N, DIM), f32),
            interpret=_INTERPRET,
        )(g, hcur, hcur, new_h, gcn_w[layer - 1])

    return hcur
